# Initial kernel scaffold; baseline (speedup 1.0000x reference)
import jax
import jax.numpy as jnp
from jax.experimental import pallas as pl


def kernel(x, neighborhood):
    raise NotImplementedError("write your pallas kernel here")


# speedup vs baseline: 5.4922x; 5.4922x over previous
import functools

import jax
import jax.numpy as jnp
from jax import lax
from jax.experimental import pallas as pl
from jax.experimental.pallas import tpu as pltpu
from jax.experimental.pallas import tpu_sc as plsc

_NC = 2
_NS = 16
_NW = _NC * _NS
_C = 128


def _cdiv(a, b):
    return (a + b - 1) // b


@functools.partial(jax.jit, static_argnames=("n", "k", "acc_rows", "rpt"))
def _sc_scatter(x, idx_r, *, n, k, acc_rows, rpt):
    d = x.shape[1]
    mesh = plsc.VectorSubcoreMesh(core_axis_name="c", subcore_axis_name="s")

    @functools.partial(
        pl.kernel,
        out_type=jax.ShapeDtypeStruct((_NC, acc_rows, d), jnp.float32),
        mesh=mesh,
        scratch_types=[
            pltpu.VMEM((2, _C), jnp.int32),
            pltpu.VMEM((2, _C), jnp.int32),
            pltpu.VMEM((_C, d), jnp.float32),
            pltpu.VMEM((_C, d), jnp.float32),
            pltpu.VMEM_SHARED((acc_rows, d), jnp.float32),
            pltpu.SemaphoreType.DMA,
            pltpu.SemaphoreType.DMA,
            pltpu.SemaphoreType.DMA,
            pltpu.SemaphoreType.DMA,
        ],
    )
    def scatter_kernel(x_hbm, idx_hbm, out_hbm,
                       ib0, ib1, rows0, rows1, acc_sh,
                       sem_g0, sem_g1, sem_i0, sem_i1):
        c = lax.axis_index("c")
        s = lax.axis_index("s")
        wid = s * _NC + c

        def zero_row(i, carry):
            zv = jnp.zeros((16,), jnp.float32)
            for j in range(d // 16):
                rows0[i, pl.ds(j * 16, 16)] = zv
            return carry

        lax.fori_loop(0, _C, zero_row, 0)
        for r in range(rpt // _C):
            pltpu.sync_copy(rows0, acc_sh.at[pl.ds(s * rpt + r * _C, _C)])

        plsc.subcore_barrier()

        pltpu.sync_copy(idx_hbm.at[wid, 0], ib0)
        pltpu.async_copy(x_hbm.at[ib0.at[0]], rows0, sem_g0)

        @pl.when(jnp.asarray(k > 1))
        def _():
            pltpu.async_copy(idx_hbm.at[wid, 1], ib1, sem_i1)

        def halfstep(j, ibp, ibq, rowsp, rowsq, semgp, semgq, semip, semiq):
            pltpu.make_async_copy(x_hbm.at[ibp.at[0]], rowsp, semgp).wait()

            @pl.when(j + 1 < k)
            def _():
                pltpu.make_async_copy(idx_hbm.at[wid, j + 1], ibq, semiq).wait()
                pltpu.async_copy(x_hbm.at[ibq.at[0]], rowsq, semgq)

            pltpu.sync_copy(rowsp, acc_sh.at[ibp.at[1]], add=True)

            @pl.when(j + 2 < k)
            def _():
                pltpu.async_copy(idx_hbm.at[wid, j + 2], ibp, semip)

        def step(j, carry):
            @pl.when(j % 2 == 0)
            def _():
                halfstep(j, ib0, ib1, rows0, rows1,
                         sem_g0, sem_g1, sem_i0, sem_i1)

            @pl.when(j % 2 == 1)
            def _():
                halfstep(j, ib1, ib0, rows1, rows0,
                         sem_g1, sem_g0, sem_i1, sem_i0)

            return carry

        lax.fori_loop(0, k, step, 0)

        plsc.subcore_barrier()

        pltpu.sync_copy(acc_sh.at[pl.ds(s * rpt, rpt)],
                        out_hbm.at[c, pl.ds(s * rpt, rpt)])

    return scatter_kernel(x, idx_r)


def _add_body(a_ref, b_ref, o_ref):
    o_ref[...] = a_ref[...] + b_ref[...]


@jax.jit
def _tc_add(a, b):
    n, d = a.shape
    blk = 1000
    return pl.pallas_call(
        _add_body,
        out_shape=jax.ShapeDtypeStruct((n, d), jnp.float32),
        grid=(_cdiv(n, blk),),
        in_specs=[
            pl.BlockSpec((blk, d), lambda i: (i, 0)),
            pl.BlockSpec((blk, d), lambda i: (i, 0)),
        ],
        out_specs=pl.BlockSpec((blk, d), lambda i: (i, 0)),
    )(a, b)


def kernel(x, neighborhood):
    n, d = x.shape
    e = neighborhood.shape[1]

    k = _cdiv(e, _NW * _C)
    cap = _NW * k * _C
    pad = cap - e

    rpt = _cdiv(n + 1, _NS * _C) * _C
    acc_rows = _NS * rpt

    src = neighborhood[0]
    dst = neighborhood[1]
    if pad:
        src = jnp.concatenate([src, jnp.zeros((pad,), jnp.int32)])
        dst = jnp.concatenate([dst, jnp.full((pad,), n, jnp.int32)])
    idx_r = jnp.stack([src.reshape(_NW, k, _C), dst.reshape(_NW, k, _C)],
                      axis=2)

    partials = _sc_scatter(x, idx_r, n=n, k=k, acc_rows=acc_rows, rpt=rpt)
    return _tc_add(partials[0, :n], partials[1, :n])


# speedup vs baseline: 6.8032x; 1.2387x over previous
import functools

import jax
import jax.numpy as jnp
from jax import lax
from jax.experimental import pallas as pl
from jax.experimental.pallas import tpu as pltpu
from jax.experimental.pallas import tpu_sc as plsc

_NC = 2
_NS = 16
_NW = _NC * _NS
_C = 120
_RB = 3
_IB = 6


def _cdiv(a, b):
    return (a + b - 1) // b


@functools.partial(jax.jit, static_argnames=("n", "k", "acc_rows", "rpt"))
def _sc_scatter(x, idx_r, *, n, k, acc_rows, rpt):
    d = x.shape[1]
    mesh = plsc.VectorSubcoreMesh(core_axis_name="c", subcore_axis_name="s")

    @functools.partial(
        pl.kernel,
        out_type=jax.ShapeDtypeStruct((_NC, acc_rows, d), jnp.float32),
        mesh=mesh,
        scratch_types=[
            [pltpu.VMEM((2, _C), jnp.int32)] * _IB,
            [pltpu.VMEM((_C, d), jnp.float32)] * _RB,
            pltpu.VMEM_SHARED((acc_rows, d), jnp.float32),
            [pltpu.SemaphoreType.DMA] * _IB,
            [pltpu.SemaphoreType.DMA] * _RB,
            [pltpu.SemaphoreType.DMA] * _RB,
        ],
    )
    def scatter_kernel(x_hbm, idx_hbm, out_hbm,
                       ibs, rows, acc_sh, sem_i, sem_g, sem_s):
        c = lax.axis_index("c")
        s = lax.axis_index("s")
        wid = s * _NC + c

        def zero_row(i, carry):
            zv = jnp.zeros((16,), jnp.float32)
            for j in range(d // 16):
                rows[0][i, pl.ds(j * 16, 16)] = zv
            return carry

        lax.fori_loop(0, _C, zero_row, 0)
        full, rem = divmod(rpt, _C)
        for r in range(full):
            pltpu.sync_copy(rows[0], acc_sh.at[pl.ds(s * rpt + r * _C, _C)])
        if rem:
            pltpu.sync_copy(rows[0].at[pl.ds(0, rem)],
                            acc_sh.at[pl.ds(s * rpt + full * _C, rem)])

        plsc.subcore_barrier()

        for b in range(min(4, k)):
            pltpu.async_copy(idx_hbm.at[wid, b], ibs[b], sem_i[b])
        pltpu.make_async_copy(idx_hbm.at[wid, 0], ibs[0], sem_i[0]).wait()
        pltpu.async_copy(x_hbm.at[ibs[0].at[0]], rows[0], sem_g[0])

        def substep(j, m):
            p = m % _RB
            pn = (m + 1) % _RB
            m1 = (m + 1) % _IB
            m4 = (m + 4) % _IB

            @pl.when(j >= 2)
            def _():
                pltpu.make_async_copy(
                    rows[pn], acc_sh.at[ibs[m4].at[1]], sem_s[pn]).wait()

            @pl.when(j + 4 < k)
            def _():
                pltpu.async_copy(idx_hbm.at[wid, j + 4], ibs[m4], sem_i[m4])

            @pl.when(j + 1 < k)
            def _():
                pltpu.make_async_copy(
                    idx_hbm.at[wid, j + 1], ibs[m1], sem_i[m1]).wait()
                pltpu.async_copy(x_hbm.at[ibs[m1].at[0]], rows[pn], sem_g[pn])

            pltpu.make_async_copy(
                x_hbm.at[ibs[m].at[0]], rows[p], sem_g[p]).wait()
            pltpu.async_copy(rows[p], acc_sh.at[ibs[m].at[1]], sem_s[p],
                             add=True)

        def step(j, carry):
            for m in range(_IB):
                @pl.when(j % _IB == m)
                def _(m=m):
                    substep(j, m)
            return carry

        lax.fori_loop(0, k, step, 0)

        for j2 in range(max(0, k - 2), k):
            p = j2 % _RB
            m = j2 % _IB
            pltpu.make_async_copy(
                rows[p], acc_sh.at[ibs[m].at[1]], sem_s[p]).wait()

        plsc.subcore_barrier()

        pltpu.sync_copy(acc_sh.at[pl.ds(s * rpt, rpt)],
                        out_hbm.at[c, pl.ds(s * rpt, rpt)])

    return scatter_kernel(x, idx_r)


def _add_body(a_ref, b_ref, o_ref):
    o_ref[...] = a_ref[...] + b_ref[...]


@jax.jit
def _tc_add(a, b):
    n, d = a.shape
    blk = 1000
    return pl.pallas_call(
        _add_body,
        out_shape=jax.ShapeDtypeStruct((n, d), jnp.float32),
        grid=(_cdiv(n, blk),),
        in_specs=[
            pl.BlockSpec((blk, d), lambda i: (i, 0)),
            pl.BlockSpec((blk, d), lambda i: (i, 0)),
        ],
        out_specs=pl.BlockSpec((blk, d), lambda i: (i, 0)),
    )(a, b)


def kernel(x, neighborhood):
    n, d = x.shape
    e = neighborhood.shape[1]

    k = _cdiv(e, _NW * _C)
    cap = _NW * k * _C
    pad = cap - e

    rpt = _cdiv(_cdiv(n + 1, _NS), 8) * 8
    acc_rows = _NS * rpt

    src = neighborhood[0]
    dst = neighborhood[1]
    if pad:
        src = jnp.concatenate([src, jnp.zeros((pad,), jnp.int32)])
        dst = jnp.concatenate([dst, jnp.full((pad,), n, jnp.int32)])
    idx_r = jnp.stack([src.reshape(_NW, k, _C), dst.reshape(_NW, k, _C)],
                      axis=2)

    partials = _sc_scatter(x, idx_r, n=n, k=k, acc_rows=acc_rows, rpt=rpt)
    return _tc_add(partials[0, :n], partials[1, :n])


# speedup vs baseline: 10.4600x; 1.5375x over previous
import functools

import jax
import jax.numpy as jnp
from jax import lax
from jax.experimental import pallas as pl
from jax.experimental.pallas import tpu as pltpu
from jax.experimental.pallas import tpu_sc as plsc

_NC = 2
_NS = 16
_NW = _NC * _NS
_C = 120
_RB = 3
_IB = 6
_FAST_CORE = 0
_FAST_SHARE = 0.70


def _cdiv(a, b):
    return (a + b - 1) // b


def _split(e):
    tch = _cdiv(e, _C)
    kf = _cdiv(int(round(tch * _FAST_SHARE)), _NS)
    ks = _cdiv(max(tch - kf * _NS, 0), _NS)
    return (kf, ks) if _FAST_CORE == 0 else (ks, kf)


@functools.partial(jax.jit, static_argnames=("n", "k0", "k1", "acc_rows",
                                             "rpt"))
def _sc_scatter(x, idx_r, *, n, k0, k1, acc_rows, rpt):
    d = x.shape[1]
    mesh = plsc.VectorSubcoreMesh(core_axis_name="c", subcore_axis_name="s")

    @functools.partial(
        pl.kernel,
        out_type=jax.ShapeDtypeStruct((_NC, acc_rows, d), jnp.float32),
        mesh=mesh,
        scratch_types=[
            [pltpu.VMEM((2, _C), jnp.int32)] * _IB,
            [pltpu.VMEM((_C, d), jnp.float32)] * _RB,
            pltpu.VMEM_SHARED((acc_rows, d), jnp.float32),
            [pltpu.SemaphoreType.DMA] * _IB,
            [pltpu.SemaphoreType.DMA] * _RB,
            [pltpu.SemaphoreType.DMA] * _RB,
        ],
    )
    def scatter_kernel(x_hbm, idx_hbm, out_hbm,
                       ibs, rows, acc_sh, sem_i, sem_g, sem_s):
        c = lax.axis_index("c")
        s = lax.axis_index("s")

        def zero_row(i, carry):
            zv = jnp.zeros((16,), jnp.float32)
            for j in range(d // 16):
                rows[0][i, pl.ds(j * 16, 16)] = zv
            return carry

        lax.fori_loop(0, _C, zero_row, 0)
        full, rem = divmod(rpt, _C)
        for r in range(full):
            pltpu.sync_copy(rows[0], acc_sh.at[pl.ds(s * rpt + r * _C, _C)])
        if rem:
            pltpu.sync_copy(rows[0].at[pl.ds(0, rem)],
                            acc_sh.at[pl.ds(s * rpt + full * _C, rem)])

        plsc.subcore_barrier()

        def pipeline(k, base):
            if k == 0:
                return
            for b in range(min(4, k)):
                pltpu.async_copy(idx_hbm.at[base + b], ibs[b], sem_i[b])
            pltpu.make_async_copy(idx_hbm.at[base], ibs[0], sem_i[0]).wait()
            pltpu.async_copy(x_hbm.at[ibs[0].at[0]], rows[0], sem_g[0])

            def substep(j, m):
                p = m % _RB
                pn = (m + 1) % _RB
                m1 = (m + 1) % _IB
                m4 = (m + 4) % _IB

                @pl.when(j >= 2)
                def _():
                    pltpu.make_async_copy(
                        rows[pn], acc_sh.at[ibs[m4].at[1]], sem_s[pn]).wait()

                @pl.when(j + 4 < k)
                def _():
                    pltpu.async_copy(idx_hbm.at[base + j + 4], ibs[m4],
                                     sem_i[m4])

                @pl.when(j + 1 < k)
                def _():
                    pltpu.make_async_copy(
                        idx_hbm.at[base + j + 1], ibs[m1], sem_i[m1]).wait()
                    pltpu.async_copy(x_hbm.at[ibs[m1].at[0]], rows[pn],
                                     sem_g[pn])

                pltpu.make_async_copy(
                    x_hbm.at[ibs[m].at[0]], rows[p], sem_g[p]).wait()
                pltpu.async_copy(rows[p], acc_sh.at[ibs[m].at[1]], sem_s[p],
                                 add=True)

            def step(j, carry):
                for m in range(_IB):
                    @pl.when(j % _IB == m)
                    def _(m=m):
                        substep(j, m)
                return carry

            lax.fori_loop(0, k, step, 0)

            for j2 in range(max(0, k - 2), k):
                p = j2 % _RB
                m = j2 % _IB
                pltpu.make_async_copy(
                    rows[p], acc_sh.at[ibs[m].at[1]], sem_s[p]).wait()

        @pl.when(c == 0)
        def _():
            pipeline(k0, s * k0)

        @pl.when(c == 1)
        def _():
            pipeline(k1, _NS * k0 + s * k1)

        plsc.subcore_barrier()

        pltpu.sync_copy(acc_sh.at[pl.ds(s * rpt, rpt)],
                        out_hbm.at[c, pl.ds(s * rpt, rpt)])

    return scatter_kernel(x, idx_r)


def _add_body(a_ref, b_ref, o_ref):
    o_ref[...] = a_ref[...] + b_ref[...]


@jax.jit
def _tc_add(a, b):
    n, d = a.shape
    blk = 1000
    return pl.pallas_call(
        _add_body,
        out_shape=jax.ShapeDtypeStruct((n, d), jnp.float32),
        grid=(_cdiv(n, blk),),
        in_specs=[
            pl.BlockSpec((blk, d), lambda i: (i, 0)),
            pl.BlockSpec((blk, d), lambda i: (i, 0)),
        ],
        out_specs=pl.BlockSpec((blk, d), lambda i: (i, 0)),
    )(a, b)


def kernel(x, neighborhood):
    n, d = x.shape
    e = neighborhood.shape[1]

    k0, k1 = _split(e)
    tch = _NS * (k0 + k1)
    cap = tch * _C
    pad = cap - e

    rpt = _cdiv(_cdiv(n + 1, _NS), 8) * 8
    acc_rows = _NS * rpt

    src = neighborhood[0]
    dst = neighborhood[1]
    if pad:
        src = jnp.concatenate([src, jnp.zeros((pad,), jnp.int32)])
        dst = jnp.concatenate([dst, jnp.full((pad,), n, jnp.int32)])
    idx_r = jnp.stack([src.reshape(tch, _C), dst.reshape(tch, _C)], axis=1)

    partials = _sc_scatter(x, idx_r, n=n, k0=k0, k1=k1, acc_rows=acc_rows,
                           rpt=rpt)
    return _tc_add(partials[0, :n], partials[1, :n])
